# baseline (speedup 1.0000x reference)
import functools

import jax
import jax.numpy as jnp
from jax import lax
from jax.experimental import pallas as pl
from jax.experimental.pallas import tpu as pltpu
from jax.experimental.pallas import tpu_sc as plsc

VOCAB = 100000
EMBED = 32
BATCH = 1024
CTX = 4


def _make_sc_pool():
    info = plsc.get_sparse_core_info()
    NC, NS, L = info.num_cores, info.num_subcores, info.num_lanes
    NW = NC * NS
    assert BATCH % NW == 0
    b_per_w = BATCH // NW
    idx_per_w = b_per_w * CTX
    mesh = plsc.VectorSubcoreMesh(core_axis_name="c", subcore_axis_name="s")

    @functools.partial(
        pl.kernel,
        mesh=mesh,
        compiler_params=pltpu.CompilerParams(use_tc_tiling_on_sc=False),
        out_type=jax.ShapeDtypeStruct((BATCH, EMBED), jnp.float32),
        scratch_types=[
            pltpu.VMEM((idx_per_w,), jnp.int32),
            pltpu.VMEM((idx_per_w, EMBED), jnp.float32),
            pltpu.VMEM((b_per_w, EMBED), jnp.float32),
            pltpu.SemaphoreType.DMA,
        ],
    )
    def sc_pool(table_hbm, idx_hbm, out_hbm, idx_v, rows_v, pooled_v, sem):
        wid = lax.axis_index("s") * NC + lax.axis_index("c")
        pltpu.sync_copy(idx_hbm.at[pl.ds(wid * idx_per_w, idx_per_w)], idx_v)
        pltpu.async_copy(table_hbm.at[idx_v], rows_v, sem).wait()
        for b in range(b_per_w):
            for c in range(EMBED // L):
                col = pl.ds(c * L, L)
                acc = rows_v[CTX * b, col]
                for k in range(1, CTX):
                    acc = acc + rows_v[CTX * b + k, col]
                pooled_v[b, col] = acc * (1.0 / CTX)
        pltpu.sync_copy(pooled_v, out_hbm.at[pl.ds(wid * b_per_w, b_per_w)])

    return sc_pool


_sc_pool = _make_sc_pool()


VBLK = 1024


def _mm_kernel(p_ref, w_ref, b_ref, o_ref):
    o_ref[...] = lax.dot_general(
        p_ref[...], w_ref[...],
        dimension_numbers=(((1,), (1,)), ((), ())),
        preferred_element_type=jnp.float32,
    ) + b_ref[...]


def _project(pooled, W_out_w, bias2d):
    grid = (pl.cdiv(VOCAB, VBLK),)
    return pl.pallas_call(
        _mm_kernel,
        grid=grid,
        in_specs=[
            pl.BlockSpec((BATCH, EMBED), lambda j: (0, 0)),
            pl.BlockSpec((VBLK, EMBED), lambda j: (j, 0)),
            pl.BlockSpec((1, VBLK), lambda j: (0, j)),
        ],
        out_specs=pl.BlockSpec((BATCH, VBLK), lambda j: (0, j)),
        out_shape=jax.ShapeDtypeStruct((BATCH, VOCAB), jnp.float32),
        compiler_params=pltpu.CompilerParams(
            dimension_semantics=("parallel",),
        ),
    )(pooled, W_out_w, bias2d)


@jax.jit
def kernel(context_words, W_in, W_out_w, W_out_b):
    idx = context_words.reshape(-1).astype(jnp.int32)
    pooled = _sc_pool(W_in, idx)
    return _project(pooled, W_out_w, W_out_b.reshape(1, VOCAB))


# speedup vs baseline: 2.0408x; 2.0408x over previous
import functools

import jax
import jax.numpy as jnp
from jax import lax
from jax.experimental import pallas as pl
from jax.experimental.pallas import tpu as pltpu
from jax.experimental.pallas import tpu_sc as plsc

VOCAB = 100000
EMBED = 32
BATCH = 1024
CTX = 4


def _make_sc_pool():
    info = plsc.get_sparse_core_info()
    NC, NS, L = info.num_cores, info.num_subcores, info.num_lanes
    NW = NC * NS
    assert BATCH % NW == 0
    b_per_w = BATCH // NW
    idx_per_w = b_per_w * CTX
    mesh = plsc.VectorSubcoreMesh(core_axis_name="c", subcore_axis_name="s")

    @functools.partial(
        pl.kernel,
        mesh=mesh,
        compiler_params=pltpu.CompilerParams(use_tc_tiling_on_sc=False),
        out_type=jax.ShapeDtypeStruct((BATCH, EMBED), jnp.float32),
        scratch_types=[
            pltpu.VMEM((idx_per_w,), jnp.int32),
            pltpu.VMEM((idx_per_w, EMBED), jnp.float32),
            pltpu.VMEM((b_per_w, EMBED), jnp.float32),
            pltpu.SemaphoreType.DMA,
        ],
    )
    def sc_pool(table_hbm, idx_hbm, out_hbm, idx_v, rows_v, pooled_v, sem):
        wid = lax.axis_index("s") * NC + lax.axis_index("c")
        pltpu.sync_copy(idx_hbm.at[pl.ds(wid * idx_per_w, idx_per_w)], idx_v)
        pltpu.async_copy(table_hbm.at[idx_v], rows_v, sem).wait()
        for b in range(b_per_w):
            for c in range(EMBED // L):
                col = pl.ds(c * L, L)
                acc = rows_v[CTX * b, col]
                for k in range(1, CTX):
                    acc = acc + rows_v[CTX * b + k, col]
                pooled_v[b, col] = acc * (1.0 / CTX)
        pltpu.sync_copy(pooled_v, out_hbm.at[pl.ds(wid * b_per_w, b_per_w)])

    return sc_pool


_sc_pool = _make_sc_pool()


VBLK = 2048


def _mm_kernel(w_ref, p_ref, b_ref, o_ref):
    o_ref[...] = lax.dot_general(
        w_ref[...], p_ref[...],
        dimension_numbers=(((1,), (1,)), ((), ())),
        preferred_element_type=jnp.float32,
    ) + b_ref[...]


def _project_t(W_out_w, pooled, bias_col):
    grid = (pl.cdiv(VOCAB, VBLK),)
    return pl.pallas_call(
        _mm_kernel,
        grid=grid,
        in_specs=[
            pl.BlockSpec((VBLK, EMBED), lambda j: (j, 0)),
            pl.BlockSpec((BATCH, EMBED), lambda j: (0, 0)),
            pl.BlockSpec((VBLK, 1), lambda j: (j, 0)),
        ],
        out_specs=pl.BlockSpec((VBLK, BATCH), lambda j: (j, 0)),
        out_shape=jax.ShapeDtypeStruct((VOCAB, BATCH), jnp.float32),
        compiler_params=pltpu.CompilerParams(
            dimension_semantics=("parallel",),
        ),
    )(W_out_w, pooled, bias_col)


@jax.jit
def kernel(context_words, W_in, W_out_w, W_out_b):
    idx = context_words.reshape(-1).astype(jnp.int32)
    pooled = _sc_pool(W_in, idx)
    out_t = _project_t(W_out_w, pooled, W_out_b.reshape(VOCAB, 1))
    return out_t.T


# speedup vs baseline: 2.0807x; 1.0196x over previous
import functools

import jax
import jax.numpy as jnp
from jax import lax
from jax.experimental import pallas as pl
from jax.experimental.pallas import tpu as pltpu
from jax.experimental.pallas import tpu_sc as plsc

VOCAB = 100000
EMBED = 32
BATCH = 1024
CTX = 4


def _make_sc_pool():
    info = plsc.get_sparse_core_info()
    NC, NS, L = info.num_cores, info.num_subcores, info.num_lanes
    NW = NC * NS
    assert BATCH % NW == 0
    b_per_w = BATCH // NW
    idx_per_w = b_per_w * CTX
    mesh = plsc.VectorSubcoreMesh(core_axis_name="c", subcore_axis_name="s")

    @functools.partial(
        pl.kernel,
        mesh=mesh,
        compiler_params=pltpu.CompilerParams(use_tc_tiling_on_sc=False),
        out_type=jax.ShapeDtypeStruct((BATCH, EMBED), jnp.float32),
        scratch_types=[
            pltpu.VMEM((idx_per_w,), jnp.int32),
            pltpu.VMEM((idx_per_w, EMBED), jnp.float32),
            pltpu.VMEM((b_per_w, EMBED), jnp.float32),
            pltpu.SemaphoreType.DMA,
        ],
    )
    def sc_pool(table_hbm, idx_hbm, out_hbm, idx_v, rows_v, pooled_v, sem):
        wid = lax.axis_index("s") * NC + lax.axis_index("c")
        pltpu.sync_copy(idx_hbm.at[pl.ds(wid * idx_per_w, idx_per_w)], idx_v)
        pltpu.async_copy(table_hbm.at[idx_v], rows_v, sem).wait()
        for b in range(b_per_w):
            for c in range(EMBED // L):
                col = pl.ds(c * L, L)
                acc = rows_v[CTX * b, col]
                for k in range(1, CTX):
                    acc = acc + rows_v[CTX * b + k, col]
                pooled_v[b, col] = acc * (1.0 / CTX)
        pltpu.sync_copy(pooled_v, out_hbm.at[pl.ds(wid * b_per_w, b_per_w)])

    return sc_pool


_sc_pool = _make_sc_pool()


VBLK = 4096


def _mm_kernel(w_ref, p_ref, b_ref, o_ref):
    o_ref[...] = lax.dot_general(
        w_ref[...], p_ref[...],
        dimension_numbers=(((1,), (1,)), ((), ())),
        preferred_element_type=jnp.float32,
    ) + b_ref[...]


def _project_t(W_out_w, pooled, bias_col):
    grid = (pl.cdiv(VOCAB, VBLK),)
    return pl.pallas_call(
        _mm_kernel,
        grid=grid,
        in_specs=[
            pl.BlockSpec((VBLK, EMBED), lambda j: (j, 0)),
            pl.BlockSpec((BATCH, EMBED), lambda j: (0, 0)),
            pl.BlockSpec((VBLK, 1), lambda j: (j, 0)),
        ],
        out_specs=pl.BlockSpec((VBLK, BATCH), lambda j: (j, 0)),
        out_shape=jax.ShapeDtypeStruct((VOCAB, BATCH), jnp.float32),
        compiler_params=pltpu.CompilerParams(
            dimension_semantics=("parallel",),
        ),
    )(W_out_w, pooled, bias_col)


@jax.jit
def kernel(context_words, W_in, W_out_w, W_out_b):
    idx = context_words.reshape(-1).astype(jnp.int32)
    pooled = _sc_pool(W_in, idx)
    out_t = _project_t(W_out_w, pooled, W_out_b.reshape(VOCAB, 1))
    return out_t.T


# speedup vs baseline: 2.3743x; 1.1411x over previous
import functools

import jax
import jax.numpy as jnp
from jax import lax
from jax.experimental import pallas as pl
from jax.experimental.pallas import tpu as pltpu
from jax.experimental.pallas import tpu_sc as plsc

VOCAB = 100000
EMBED = 32
BATCH = 1024
CTX = 4


def _make_sc_pool():
    info = plsc.get_sparse_core_info()
    NC, NS, L = info.num_cores, info.num_subcores, info.num_lanes
    NW = NC * NS
    assert BATCH % NW == 0
    b_per_w = BATCH // NW
    idx_per_w = b_per_w * CTX
    mesh = plsc.VectorSubcoreMesh(core_axis_name="c", subcore_axis_name="s")

    @functools.partial(
        pl.kernel,
        mesh=mesh,
        compiler_params=pltpu.CompilerParams(use_tc_tiling_on_sc=False),
        out_type=jax.ShapeDtypeStruct((BATCH, EMBED), jnp.float32),
        scratch_types=[
            pltpu.VMEM((idx_per_w,), jnp.int32),
            pltpu.VMEM((idx_per_w, EMBED), jnp.float32),
            pltpu.VMEM((b_per_w, EMBED), jnp.float32),
            pltpu.SemaphoreType.DMA,
        ],
    )
    def sc_pool(table_hbm, idx_hbm, out_hbm, idx_v, rows_v, pooled_v, sem):
        wid = lax.axis_index("s") * NC + lax.axis_index("c")
        pltpu.sync_copy(idx_hbm.at[pl.ds(wid * idx_per_w, idx_per_w)], idx_v)
        pltpu.async_copy(table_hbm.at[idx_v], rows_v, sem).wait()
        for b in range(b_per_w):
            for c in range(EMBED // L):
                col = pl.ds(c * L, L)
                acc = rows_v[CTX * b, col]
                for k in range(1, CTX):
                    acc = acc + rows_v[CTX * b + k, col]
                pooled_v[b, col] = acc * (1.0 / CTX)
        pltpu.sync_copy(pooled_v, out_hbm.at[pl.ds(wid * b_per_w, b_per_w)])

    return sc_pool


_sc_pool = _make_sc_pool()


VBLK = 2048


def _mm_kernel(wt_ref, p_ref, b_ref, o_ref):
    o_ref[...] = lax.dot_general(
        wt_ref[...], p_ref[...],
        dimension_numbers=(((0,), (1,)), ((), ())),
        preferred_element_type=jnp.float32,
    ) + b_ref[...]


def _project_t(W_out_w_t, pooled, bias_col):
    grid = (pl.cdiv(VOCAB, VBLK),)
    return pl.pallas_call(
        _mm_kernel,
        grid=grid,
        in_specs=[
            pl.BlockSpec((EMBED, VBLK), lambda j: (0, j)),
            pl.BlockSpec((BATCH, EMBED), lambda j: (0, 0)),
            pl.BlockSpec((VBLK, 1), lambda j: (j, 0)),
        ],
        out_specs=pl.BlockSpec((VBLK, BATCH), lambda j: (j, 0)),
        out_shape=jax.ShapeDtypeStruct((VOCAB, BATCH), jnp.float32),
        compiler_params=pltpu.CompilerParams(
            dimension_semantics=("parallel",),
        ),
    )(W_out_w_t, pooled, bias_col)


@jax.jit
def kernel(context_words, W_in, W_out_w, W_out_b):
    idx = context_words.reshape(-1).astype(jnp.int32)
    pooled = _sc_pool(W_in, idx)
    out_t = _project_t(W_out_w.T, pooled, W_out_b.reshape(VOCAB, 1))
    return out_t.T


# speedup vs baseline: 2.3873x; 1.0055x over previous
import functools

import jax
import jax.numpy as jnp
from jax import lax
from jax.experimental import pallas as pl
from jax.experimental.pallas import tpu as pltpu
from jax.experimental.pallas import tpu_sc as plsc

VOCAB = 100000
EMBED = 32
BATCH = 1024
CTX = 4


def _make_sc_pool():
    info = plsc.get_sparse_core_info()
    NC, NS, L = info.num_cores, info.num_subcores, info.num_lanes
    NW = NC * NS
    assert BATCH % NW == 0
    b_per_w = BATCH // NW
    idx_per_w = b_per_w * CTX
    mesh = plsc.VectorSubcoreMesh(core_axis_name="c", subcore_axis_name="s")

    @functools.partial(
        pl.kernel,
        mesh=mesh,
        compiler_params=pltpu.CompilerParams(use_tc_tiling_on_sc=False),
        out_type=jax.ShapeDtypeStruct((BATCH, EMBED), jnp.float32),
        scratch_types=[
            pltpu.VMEM((idx_per_w,), jnp.int32),
            pltpu.VMEM((idx_per_w, EMBED), jnp.float32),
            pltpu.VMEM((b_per_w, EMBED), jnp.float32),
            pltpu.SemaphoreType.DMA,
        ],
    )
    def sc_pool(table_hbm, idx_hbm, out_hbm, idx_v, rows_v, pooled_v, sem):
        wid = lax.axis_index("s") * NC + lax.axis_index("c")
        pltpu.sync_copy(idx_hbm.at[pl.ds(wid * idx_per_w, idx_per_w)], idx_v)
        pltpu.async_copy(table_hbm.at[idx_v], rows_v, sem).wait()
        for b in range(b_per_w):
            for c in range(EMBED // L):
                col = pl.ds(c * L, L)
                acc = rows_v[CTX * b, col]
                for k in range(1, CTX):
                    acc = acc + rows_v[CTX * b + k, col]
                pooled_v[b, col] = acc * (1.0 / CTX)
        pltpu.sync_copy(pooled_v, out_hbm.at[pl.ds(wid * b_per_w, b_per_w)])

    return sc_pool


_sc_pool = _make_sc_pool()


VBLK = 2048
NBLK = pl.cdiv(VOCAB, VBLK)
TAIL = VOCAB - (NBLK - 1) * VBLK
NBUF = 3


def _mm_kernel(wt_ref, p_ref, b_ref, o_hbm, o_buf, sems):
    j = pl.program_id(0)
    slot = lax.rem(j, NBUF)

    @pl.when(j >= NBUF)
    def _wait_prev():
        pltpu.make_async_copy(
            o_buf.at[slot],
            o_hbm.at[pl.ds((j - NBUF) * VBLK, VBLK), :],
            sems.at[slot],
        ).wait()

    o_buf[slot] = lax.dot_general(
        wt_ref[...], p_ref[...],
        dimension_numbers=(((0,), (1,)), ((), ())),
        preferred_element_type=jnp.float32,
    ) + b_ref[...]

    @pl.when(j < NBLK - 1)
    def _start_full():
        pltpu.make_async_copy(
            o_buf.at[slot],
            o_hbm.at[pl.ds(j * VBLK, VBLK), :],
            sems.at[slot],
        ).start()

    @pl.when(j == NBLK - 1)
    def _last():
        pltpu.make_async_copy(
            o_buf.at[slot, pl.ds(0, TAIL)],
            o_hbm.at[pl.ds((NBLK - 1) * VBLK, TAIL), :],
            sems.at[slot],
        ).start()
        for k in range(NBUF):
            g = NBLK - NBUF + k
            s = g % NBUF
            if g == NBLK - 1:
                pltpu.make_async_copy(
                    o_buf.at[s, pl.ds(0, TAIL)],
                    o_hbm.at[pl.ds(g * VBLK, TAIL), :],
                    sems.at[s],
                ).wait()
            else:
                pltpu.make_async_copy(
                    o_buf.at[s],
                    o_hbm.at[pl.ds(g * VBLK, VBLK), :],
                    sems.at[s],
                ).wait()


def _project_t(W_out_w_t, pooled, bias_col):
    return pl.pallas_call(
        _mm_kernel,
        grid=(NBLK,),
        in_specs=[
            pl.BlockSpec((EMBED, VBLK), lambda j: (0, j)),
            pl.BlockSpec((BATCH, EMBED), lambda j: (0, 0)),
            pl.BlockSpec((VBLK, 1), lambda j: (j, 0)),
        ],
        out_specs=pl.BlockSpec(memory_space=pl.ANY),
        out_shape=jax.ShapeDtypeStruct((VOCAB, BATCH), jnp.float32),
        scratch_shapes=[
            pltpu.VMEM((NBUF, VBLK, BATCH), jnp.float32),
            pltpu.SemaphoreType.DMA((NBUF,)),
        ],
        compiler_params=pltpu.CompilerParams(
            dimension_semantics=("arbitrary",),
        ),
    )(W_out_w_t, pooled, bias_col)


@jax.jit
def kernel(context_words, W_in, W_out_w, W_out_b):
    idx = context_words.reshape(-1).astype(jnp.int32)
    pooled = _sc_pool(W_in, idx)
    out_t = _project_t(W_out_w.T, pooled, W_out_b.reshape(VOCAB, 1))
    return out_t.T
